# baseline (speedup 1.0000x reference)
import functools

import jax
import jax.numpy as jnp
from jax.experimental import pallas as pl
from jax.experimental.pallas import tpu as pltpu

MARGIN_ = 1.0
B_ = 4096
RB_ = 512
NB_ = B_ // RB_


def _triplet_block_kernel(emb_row_ref, emb_all_ref, tgt_row_ref, tgt_col_ref,
                          out_ref, acc_ref, cdiff_ref, sqall_ref):
    i = pl.program_id(0)

    @pl.when(i == 0)
    def _init():
        acc_ref[0] = 0.0
        acc_ref[1] = 0.0
        acc_ref[2] = 0.0
        col = jax.lax.broadcasted_iota(jnp.int32, (RB_, B_), 1)
        row = jax.lax.broadcasted_iota(jnp.int32, (RB_, B_), 0)
        cdiff_ref[...] = col - row
        e_all0 = emb_all_ref[...]
        sqall_ref[...] = jnp.sum(e_all0 * e_all0, axis=1, keepdims=True).T

    e_row = emb_row_ref[...]
    e_all = emb_all_ref[...]
    sq_row = jnp.sum(e_row * e_row, axis=1, keepdims=True)
    sq_all = sqall_ref[...]
    gram = jax.lax.dot_general(
        e_row, e_all,
        dimension_numbers=(((1,), (1,)), ((), ())),
        preferred_element_type=jnp.float32,
    )
    S = jnp.maximum(sq_row + sq_all - 2.0 * gram, 0.0)

    t_all = tgt_row_ref[...]
    t_row = tgt_col_ref[...]
    same = t_row == t_all

    s_neg = jnp.where(same, jnp.inf, S)
    s_an = jnp.min(s_neg, axis=1, keepdims=True)

    t = jnp.sqrt(s_an) - MARGIN_
    thr = jnp.where(t < 0.0, -1.0, t * t)
    cond = S > thr

    pair = same & (cdiff_ref[...] > i * RB_)
    tri = pair & cond

    x = S - (s_an - MARGIN_)
    losses = jnp.maximum(x, 0.0)
    zero = jnp.zeros_like(S)
    one = jnp.ones_like(S)
    loss_part = jnp.sum(jnp.where(pair, losses, zero))
    cnt_part = jnp.sum(jnp.where(tri, one, zero))
    acc_part = jnp.sum(jnp.where(tri & (x < MARGIN_), one, zero))

    acc_ref[0] += loss_part
    acc_ref[1] += cnt_part
    acc_ref[2] += acc_part

    @pl.when(i == NB_ - 1)
    def _finish():
        cnt = acc_ref[1]
        out_ref[0] = acc_ref[0] / cnt
        out_ref[1] = acc_ref[2] / cnt


@functools.partial(jax.jit, static_argnames=())
def _run(embeddings, targets):
    tgt_row = targets.astype(jnp.int32).reshape(1, B_)
    tgt_col = targets.astype(jnp.int32).reshape(B_, 1)
    out = pl.pallas_call(
        _triplet_block_kernel,
        grid=(NB_,),
        in_specs=[
            pl.BlockSpec((RB_, 128), lambda i: (i, 0)),
            pl.BlockSpec((B_, 128), lambda i: (0, 0)),
            pl.BlockSpec((1, B_), lambda i: (0, 0)),
            pl.BlockSpec((RB_, 1), lambda i: (i, 0)),
        ],
        out_specs=pl.BlockSpec(memory_space=pltpu.SMEM),
        out_shape=jax.ShapeDtypeStruct((2,), jnp.float32),
        scratch_shapes=[
            pltpu.SMEM((3,), jnp.float32),
            pltpu.VMEM((RB_, B_), jnp.int32),
            pltpu.VMEM((1, B_), jnp.float32),
        ],
    )(embeddings, embeddings, tgt_row, tgt_col)
    return out[0], out[1]


def kernel(embeddings, targets):
    loss, accuracy = _run(embeddings, targets)
    return loss.reshape(()), accuracy.reshape(())


# speedup vs baseline: 1.0019x; 1.0019x over previous
import functools

import jax
import jax.numpy as jnp
from jax.experimental import pallas as pl
from jax.experimental.pallas import tpu as pltpu

MARGIN_ = 1.0
B_ = 4096
RB_ = 512
NB_ = B_ // RB_


def _triplet_block_kernel(emb_row_ref, emb_all_ref, tgt_row_ref, tgt_col_ref,
                          out_ref, acc_ref, sqall_ref):
    i = pl.program_id(0)

    @pl.when(i == 0)
    def _init():
        acc_ref[0] = 0.0
        acc_ref[1] = 0.0
        acc_ref[2] = 0.0
        e_all0 = emb_all_ref[...]
        sqall_ref[...] = jnp.sum(e_all0 * e_all0, axis=1, keepdims=True).T

    e_row = emb_row_ref[...]
    e_all = emb_all_ref[...]
    sq_row = jnp.sum(e_row * e_row, axis=1, keepdims=True)
    sq_all = sqall_ref[...]
    gram = jax.lax.dot_general(
        e_row, e_all,
        dimension_numbers=(((1,), (1,)), ((), ())),
        preferred_element_type=jnp.float32,
    )
    S = jnp.maximum(sq_row + sq_all - 2.0 * gram, 0.0)

    t_all = tgt_row_ref[...]
    t_row = tgt_col_ref[...]
    same = t_row == t_all

    s_neg = jnp.where(same, jnp.inf, S)
    s_an = jnp.min(s_neg, axis=1, keepdims=True)

    t = jnp.sqrt(s_an) - MARGIN_
    thr = jnp.where(t < 0.0, -1.0, t * t)
    cond = S > thr

    col = jax.lax.broadcasted_iota(jnp.int32, (RB_, B_), 1)
    row = jax.lax.broadcasted_iota(jnp.int32, (RB_, 1), 0) + i * RB_
    pair = same & (col > row)
    tri = pair & cond

    x = S - (s_an - MARGIN_)
    losses = jnp.maximum(x, 0.0)
    zero = jnp.zeros_like(S)
    one = jnp.ones_like(S)
    loss_part = jnp.sum(jnp.where(pair, losses, zero))
    cnt_part = jnp.sum(jnp.where(tri, one, zero))
    acc_part = jnp.sum(jnp.where(tri & (x < MARGIN_), one, zero))

    acc_ref[0] += loss_part
    acc_ref[1] += cnt_part
    acc_ref[2] += acc_part

    @pl.when(i == NB_ - 1)
    def _finish():
        cnt = acc_ref[1]
        out_ref[0] = acc_ref[0] / cnt
        out_ref[1] = acc_ref[2] / cnt


@functools.partial(jax.jit, static_argnames=())
def _run(embeddings, targets):
    tgt_row = targets.astype(jnp.int32).reshape(1, B_)
    tgt_col = targets.astype(jnp.int32).reshape(B_, 1)
    out = pl.pallas_call(
        _triplet_block_kernel,
        grid=(NB_,),
        in_specs=[
            pl.BlockSpec((RB_, 128), lambda i: (i, 0)),
            pl.BlockSpec((B_, 128), lambda i: (0, 0)),
            pl.BlockSpec((1, B_), lambda i: (0, 0)),
            pl.BlockSpec((RB_, 1), lambda i: (i, 0)),
        ],
        out_specs=pl.BlockSpec(memory_space=pltpu.SMEM),
        out_shape=jax.ShapeDtypeStruct((2,), jnp.float32),
        scratch_shapes=[
            pltpu.SMEM((3,), jnp.float32),
            pltpu.VMEM((1, B_), jnp.float32),
        ],
    )(embeddings, embeddings, tgt_row, tgt_col)
    return out[0], out[1]


def kernel(embeddings, targets):
    loss, accuracy = _run(embeddings, targets)
    return loss.reshape(()), accuracy.reshape(())


# speedup vs baseline: 1.1796x; 1.1775x over previous
import functools

import jax
import jax.numpy as jnp
from jax.experimental import pallas as pl
from jax.experimental.pallas import tpu as pltpu

MARGIN_ = 1.0
BIG_ = float(2 ** 60)
B_ = 4096
RB_ = 512
NB_ = B_ // RB_


def _triplet_block_kernel(emb_ref, tgt_col_ref, out_ref,
                          acc_ref, sqall_ref, oh_ref, lt_ref):
    i = pl.program_id(0)

    @pl.when(i == 0)
    def _init():
        acc_ref[0] = 0.0
        acc_ref[1] = 0.0
        acc_ref[2] = 0.0
        e_all0 = emb_ref[...]
        sqall_ref[...] = jnp.sum(e_all0 * e_all0, axis=1, keepdims=True).T
        lane = jax.lax.broadcasted_iota(jnp.int32, (B_, 128), 1)
        oh_ref[...] = jnp.where(lane == tgt_col_ref[...], 1.0, 0.0)
        lc = jax.lax.broadcasted_iota(jnp.int32, (RB_, RB_), 1)
        lr = jax.lax.broadcasted_iota(jnp.int32, (RB_, RB_), 0)
        lt_ref[...] = jnp.where(lc > lr, 1.0, 0.0)

    e_all = emb_ref[...]
    e_row = emb_ref[pl.ds(i * RB_, RB_), :]
    m2e = e_row * -2.0
    oh_all = oh_ref[...]
    oh_row = oh_ref[pl.ds(i * RB_, RB_), :]

    dims = (((1,), (1,)), ((), ()))
    dot = functools.partial(
        jax.lax.dot_general, dimension_numbers=dims,
        preferred_element_type=jnp.float32)

    P = dot(m2e, e_all) + sqall_ref[...]
    same_f = dot(oh_row, oh_all)
    notsame = 1.0 - same_f

    s_neg = same_f * BIG_ + P
    rowmin = jnp.min(s_neg, axis=1, keepdims=True)
    sq_row = jnp.sum(e_row * e_row, axis=1, keepdims=True)
    s_an = jnp.maximum(rowmin + sq_row, 0.0)

    t = jnp.sqrt(s_an) - MARGIN_
    thr = jnp.where(t < 0.0, -1.0, t * t)
    c1 = s_an - MARGIN_ - sq_row
    c2 = thr - sq_row
    c3 = s_an - sq_row

    y = notsame * -BIG_ + P
    relu_f = jnp.maximum(y - c1, 0.0)
    cnt_f = jnp.where(y > c2, 1.0, 0.0)
    ge_f = jnp.where(y >= c3, 1.0, 0.0)

    colid = jax.lax.broadcasted_iota(jnp.int32, (1, B_), 1)
    rv = jnp.where(colid >= (i + 1) * RB_, 1.0, 0.0)
    loss_rows = dot(relu_f, rv)
    cnt_rows = dot(cnt_f, rv)
    ge_rows = dot(ge_f, rv)

    Pd = dot(m2e, e_row) + sqall_ref[0:1, pl.ds(i * RB_, RB_)]
    samed = dot(oh_row, oh_row)
    yd = (1.0 - samed) * -BIG_ + Pd
    lt = lt_ref[...]
    relu_d = jnp.maximum(yd - c1, 0.0) * lt
    cnt_d = jnp.where(yd > c2, lt, 0.0)
    ge_d = jnp.where(yd >= c3, lt, 0.0)
    onesd = jnp.ones((1, RB_), jnp.float32)
    loss_rows += dot(relu_d, onesd)
    cnt_rows += dot(cnt_d, onesd)
    ge_rows += dot(ge_d, onesd)

    acc_ref[0] += jnp.sum(loss_rows)
    acc_ref[1] += jnp.sum(cnt_rows)
    acc_ref[2] += jnp.sum(ge_rows)

    @pl.when(i == NB_ - 1)
    def _finish():
        cnt = acc_ref[1]
        out_ref[0] = acc_ref[0] / cnt
        out_ref[1] = (cnt - acc_ref[2]) / cnt


@jax.jit
def _run(embeddings, targets):
    tgt_col = targets.astype(jnp.int32).reshape(B_, 1)
    out = pl.pallas_call(
        _triplet_block_kernel,
        grid=(NB_,),
        in_specs=[
            pl.BlockSpec((B_, 128), lambda i: (0, 0)),
            pl.BlockSpec((B_, 1), lambda i: (0, 0)),
        ],
        out_specs=pl.BlockSpec(memory_space=pltpu.SMEM),
        out_shape=jax.ShapeDtypeStruct((2,), jnp.float32),
        scratch_shapes=[
            pltpu.SMEM((3,), jnp.float32),
            pltpu.VMEM((1, B_), jnp.float32),
            pltpu.VMEM((B_, 128), jnp.float32),
            pltpu.VMEM((RB_, RB_), jnp.float32),
        ],
    )(embeddings, tgt_col)
    return out[0], out[1]


def kernel(embeddings, targets):
    loss, accuracy = _run(embeddings, targets)
    return loss.reshape(()), accuracy.reshape(())
